# Initial kernel scaffold; baseline (speedup 1.0000x reference)
import jax
import jax.numpy as jnp
from jax.experimental import pallas as pl


def kernel(node_ids, edge_index, edge_type, batch_id, answers, corrupted, ent_table, W_rel, W_root, b_conv, lin1_W, lin1_b, lin2_W, lin2_b):
    raise NotImplementedError("write your pallas kernel here")


# speedup vs baseline: 1.1316x; 1.1316x over previous
import functools

import jax
import jax.numpy as jnp
from jax.experimental import pallas as pl
from jax.experimental.pallas import tpu as pltpu


def _dense_chain_kernel(acc_ref, inv_ref, x_ref, bid_ref, W_rel_ref,
                        W_root_ref, b_conv_ref, lin1_W_ref, lin1_b_ref,
                        lin2_W_ref, lin2_b_ref, q_ref, *, R, B):
    i = pl.program_id(0)

    @pl.when(i == 0)
    def _():
        q_ref[...] = jnp.zeros_like(q_ref)

    x = x_ref[...]
    agg = x @ W_root_ref[...]
    for r in range(R):
        scaled = acc_ref[:, r, :] * inv_ref[:, r][:, None]
        agg = agg + scaled @ W_rel_ref[r]
    h = jax.nn.relu(agg + b_conv_ref[...])
    h = jax.nn.relu(h @ lin1_W_ref[...] + lin1_b_ref[...])
    h = h @ lin2_W_ref[...] + lin2_b_ref[...]

    bid = bid_ref[:, 0]
    bn = bid.shape[0]
    onehot = jnp.where(
        bid[:, None] == jax.lax.broadcasted_iota(jnp.int32, (bn, B), 1),
        1.0, 0.0)
    q_ref[...] += jax.lax.dot_general(
        onehot, h, (((0,), (0,)), ((), ())),
        preferred_element_type=jnp.float32)


def _score_kernel(q_ref, a_ref, c_ref, loss_ref, g_ref, c_out_ref):
    q = q_ref[...]
    a = a_ref[...]
    c = c_ref[...]
    qn = jnp.sqrt(jnp.sum(q * q, axis=1, keepdims=True))
    golden = jnp.sum(q * a, axis=1, keepdims=True) / (
        qn * jnp.sqrt(jnp.sum(a * a, axis=1, keepdims=True)))
    corrupt = jnp.sum(q * c, axis=1, keepdims=True) / (
        qn * jnp.sqrt(jnp.sum(c * c, axis=1, keepdims=True)))
    g_ref[...] = golden
    c_out_ref[...] = corrupt
    loss_ref[...] = jnp.maximum(0.0, -(golden - corrupt) + 1.0)


def kernel(node_ids, edge_index, edge_type, batch_id, answers, corrupted,
           ent_table, W_rel, W_root, b_conv, lin1_W, lin1_b, lin2_W, lin2_b):
    N = node_ids.shape[0]
    R, D, H = W_rel.shape
    B = answers.shape[0]

    x = jnp.take(ent_table, node_ids, axis=0)
    src, dst = edge_index[0], edge_index[1]
    g = dst * R + edge_type
    cnt = jnp.zeros((N * R,), jnp.float32).at[g].add(1.0)
    acc = jnp.zeros((N * R, D), jnp.float32).at[g].add(x[src])
    inv = (1.0 / jnp.clip(cnt, 1.0, None)).reshape(N, R)
    acc3 = acc.reshape(N, R, D)

    bn = 1000
    grid = N // bn
    q = pl.pallas_call(
        functools.partial(_dense_chain_kernel, R=R, B=B),
        grid=(grid,),
        in_specs=[
            pl.BlockSpec((bn, R, D), lambda i: (i, 0, 0)),
            pl.BlockSpec((bn, R), lambda i: (i, 0)),
            pl.BlockSpec((bn, D), lambda i: (i, 0)),
            pl.BlockSpec((bn, 1), lambda i: (i, 0)),
            pl.BlockSpec((R, D, H), lambda i: (0, 0, 0)),
            pl.BlockSpec((D, H), lambda i: (0, 0)),
            pl.BlockSpec((1, H), lambda i: (0, 0)),
            pl.BlockSpec((H, D), lambda i: (0, 0)),
            pl.BlockSpec((1, D), lambda i: (0, 0)),
            pl.BlockSpec((D, D), lambda i: (0, 0)),
            pl.BlockSpec((1, D), lambda i: (0, 0)),
        ],
        out_specs=pl.BlockSpec((B, D), lambda i: (0, 0)),
        out_shape=jax.ShapeDtypeStruct((B, D), jnp.float32),
    )(acc3, inv, x, batch_id.reshape(N, 1).astype(jnp.int32), W_rel,
      W_root, b_conv.reshape(1, H), lin1_W, lin1_b.reshape(1, D),
      lin2_W, lin2_b.reshape(1, D))

    a_emb = jnp.take(ent_table, answers, axis=0)
    c_emb = jnp.take(ent_table, corrupted, axis=0)
    loss, golden, corrupt = pl.pallas_call(
        _score_kernel,
        out_shape=[jax.ShapeDtypeStruct((B, 1), jnp.float32)] * 3,
    )(q, a_emb, c_emb)
    return (loss.reshape(B), golden.reshape(B), corrupt.reshape(B))


# speedup vs baseline: 1.4235x; 1.2580x over previous
import functools

import jax
import jax.numpy as jnp
from jax.experimental import pallas as pl
from jax.experimental.pallas import tpu as pltpu


def _dense_chain_kernel(acc_ref, inv_ref, x_ref, bid_ref, W_rel_ref,
                        W_root_ref, b_conv_ref, lin1_W_ref, lin1_b_ref,
                        lin2_W_ref, lin2_b_ref, q_ref, *, R, B):
    i = pl.program_id(0)

    @pl.when(i == 0)
    def _():
        q_ref[...] = jnp.zeros_like(q_ref)

    x = x_ref[...]
    agg = x @ W_root_ref[...]
    for r in range(R):
        scaled = acc_ref[:, r, :] * inv_ref[:, r][:, None]
        agg = agg + scaled @ W_rel_ref[r]
    h = jax.nn.relu(agg + b_conv_ref[...])
    h = jax.nn.relu(h @ lin1_W_ref[...] + lin1_b_ref[...])
    h = h @ lin2_W_ref[...] + lin2_b_ref[...]

    bid = bid_ref[:, 0]
    bn = bid.shape[0]
    onehot = jnp.where(
        bid[:, None] == jax.lax.broadcasted_iota(jnp.int32, (bn, B), 1),
        1.0, 0.0)
    q_ref[...] += jax.lax.dot_general(
        onehot, h, (((0,), (0,)), ((), ())),
        preferred_element_type=jnp.float32)


def _score_kernel(q_ref, a_ref, c_ref, loss_ref, g_ref, c_out_ref):
    q = q_ref[...]
    a = a_ref[...]
    c = c_ref[...]
    qn = jnp.sqrt(jnp.sum(q * q, axis=1, keepdims=True))
    golden = jnp.sum(q * a, axis=1, keepdims=True) / (
        qn * jnp.sqrt(jnp.sum(a * a, axis=1, keepdims=True)))
    corrupt = jnp.sum(q * c, axis=1, keepdims=True) / (
        qn * jnp.sqrt(jnp.sum(c * c, axis=1, keepdims=True)))
    g_ref[...] = golden
    c_out_ref[...] = corrupt
    loss_ref[...] = jnp.maximum(0.0, -(golden - corrupt) + 1.0)


def kernel(node_ids, edge_index, edge_type, batch_id, answers, corrupted,
           ent_table, W_rel, W_root, b_conv, lin1_W, lin1_b, lin2_W, lin2_b):
    N = node_ids.shape[0]
    R, D, H = W_rel.shape
    B = answers.shape[0]

    x = jnp.take(ent_table, node_ids, axis=0)
    src, dst = edge_index[0], edge_index[1]
    g = dst * R + edge_type
    upd = jnp.concatenate(
        [x[src], jnp.ones((src.shape[0], 1), jnp.float32)], axis=1)
    acc = jnp.zeros((N * R, D + 1), jnp.float32).at[g].add(upd)
    inv = (1.0 / jnp.clip(acc[:, D], 1.0, None)).reshape(N, R)
    acc3 = acc[:, :D].reshape(N, R, D)

    bn = 1000
    grid = N // bn
    q = pl.pallas_call(
        functools.partial(_dense_chain_kernel, R=R, B=B),
        grid=(grid,),
        in_specs=[
            pl.BlockSpec((bn, R, D), lambda i: (i, 0, 0)),
            pl.BlockSpec((bn, R), lambda i: (i, 0)),
            pl.BlockSpec((bn, D), lambda i: (i, 0)),
            pl.BlockSpec((bn, 1), lambda i: (i, 0)),
            pl.BlockSpec((R, D, H), lambda i: (0, 0, 0)),
            pl.BlockSpec((D, H), lambda i: (0, 0)),
            pl.BlockSpec((1, H), lambda i: (0, 0)),
            pl.BlockSpec((H, D), lambda i: (0, 0)),
            pl.BlockSpec((1, D), lambda i: (0, 0)),
            pl.BlockSpec((D, D), lambda i: (0, 0)),
            pl.BlockSpec((1, D), lambda i: (0, 0)),
        ],
        out_specs=pl.BlockSpec((B, D), lambda i: (0, 0)),
        out_shape=jax.ShapeDtypeStruct((B, D), jnp.float32),
    )(acc3, inv, x, batch_id.reshape(N, 1).astype(jnp.int32), W_rel,
      W_root, b_conv.reshape(1, H), lin1_W, lin1_b.reshape(1, D),
      lin2_W, lin2_b.reshape(1, D))

    a_emb = jnp.take(ent_table, answers, axis=0)
    c_emb = jnp.take(ent_table, corrupted, axis=0)
    loss, golden, corrupt = pl.pallas_call(
        _score_kernel,
        out_shape=[jax.ShapeDtypeStruct((B, 1), jnp.float32)] * 3,
    )(q, a_emb, c_emb)
    return (loss.reshape(B), golden.reshape(B), corrupt.reshape(B))


# speedup vs baseline: 1.7937x; 1.2601x over previous
import functools

import jax
import jax.numpy as jnp
from jax import lax
from jax.experimental import pallas as pl
from jax.experimental.pallas import tpu as pltpu
from jax.experimental.pallas import tpu_sc as plsc

_DP = 64


def _gather_updates(x_pad, src):
    E = src.shape[0]
    NW = 32
    per_w = E // NW
    sub = 1000
    nsub = per_w // sub
    mesh = plsc.VectorSubcoreMesh(core_axis_name="c", subcore_axis_name="s")

    @functools.partial(
        pl.kernel,
        out_type=jax.ShapeDtypeStruct((E, _DP), jnp.float32),
        mesh=mesh,
        scratch_types=[
            pltpu.VMEM((sub,), jnp.int32),
            pltpu.VMEM((sub, _DP), jnp.float32),
            pltpu.SemaphoreType.DMA,
        ],
        compiler_params=pltpu.CompilerParams(use_tc_tiling_on_sc=False),
    )
    def k(src_hbm, table_hbm, out_hbm, idx_v, rows_v, sem):
        wid = lax.axis_index("s") * 2 + lax.axis_index("c")
        base = wid * per_w

        def body(j, carry):
            off = base + j * sub
            pltpu.sync_copy(src_hbm.at[pl.ds(off, sub)], idx_v)
            pltpu.async_copy(table_hbm.at[idx_v], rows_v, sem).wait()
            pltpu.sync_copy(rows_v, out_hbm.at[pl.ds(off, sub)])
            return carry

        lax.fori_loop(0, nsub, body, 0)

    return k(src, x_pad)


def _dense_chain_kernel(acc_ref, x_ref, bid_ref, W_rel_ref, W_root_ref,
                        b_conv_ref, lin1_W_ref, lin1_b_ref, lin2_W_ref,
                        lin2_b_ref, q_ref, *, R, B, D):
    i = pl.program_id(0)

    @pl.when(i == 0)
    def _():
        q_ref[...] = jnp.zeros_like(q_ref)

    acc = acc_ref[...]
    inv = 1.0 / jnp.clip(acc[:, :, D], 1.0, None)
    x = x_ref[...]
    agg = x @ W_root_ref[...]
    for r in range(R):
        scaled = acc[:, r, :] * inv[:, r][:, None]
        agg = agg + scaled @ W_rel_ref[r]
    h = jax.nn.relu(agg + b_conv_ref[...])
    h = jax.nn.relu(h @ lin1_W_ref[...] + lin1_b_ref[...])
    h = h @ lin2_W_ref[...] + lin2_b_ref[...]

    bid = bid_ref[:, 0]
    bn = bid.shape[0]
    onehot = jnp.where(
        bid[:, None] == jax.lax.broadcasted_iota(jnp.int32, (bn, B), 1),
        1.0, 0.0)
    q_ref[...] += jax.lax.dot_general(
        onehot, h, (((0,), (0,)), ((), ())),
        preferred_element_type=jnp.float32)


def _score_kernel(q_ref, a_ref, c_ref, loss_ref, g_ref, c_out_ref):
    q = q_ref[...]
    a = a_ref[...]
    c = c_ref[...]
    qn = jnp.sqrt(jnp.sum(q * q, axis=1, keepdims=True))
    golden = jnp.sum(q * a, axis=1, keepdims=True) / (
        qn * jnp.sqrt(jnp.sum(a * a, axis=1, keepdims=True)))
    corrupt = jnp.sum(q * c, axis=1, keepdims=True) / (
        qn * jnp.sqrt(jnp.sum(c * c, axis=1, keepdims=True)))
    g_ref[...] = golden
    c_out_ref[...] = corrupt
    loss_ref[...] = jnp.maximum(0.0, -(golden - corrupt) + 1.0)


def kernel(node_ids, edge_index, edge_type, batch_id, answers, corrupted,
           ent_table, W_rel, W_root, b_conv, lin1_W, lin1_b, lin2_W, lin2_b):
    N = node_ids.shape[0]
    R, D, H = W_rel.shape
    B = answers.shape[0]

    x = jnp.take(ent_table, node_ids, axis=0)
    x_pad = jnp.zeros((N, _DP), jnp.float32)
    x_pad = x_pad.at[:, :D].set(x).at[:, D].set(1.0)
    src, dst = edge_index[0], edge_index[1]
    g = dst * R + edge_type

    upd = _gather_updates(x_pad, src.astype(jnp.int32))
    acc = jnp.zeros((N * R, _DP), jnp.float32).at[g].add(upd)
    acc3 = acc.reshape(N, R, _DP)

    W_rel_pad = jnp.zeros((R, _DP, H), jnp.float32).at[:, :D, :].set(W_rel)

    bn = 1000
    grid = N // bn
    q = pl.pallas_call(
        functools.partial(_dense_chain_kernel, R=R, B=B, D=D),
        grid=(grid,),
        in_specs=[
            pl.BlockSpec((bn, R, _DP), lambda i: (i, 0, 0)),
            pl.BlockSpec((bn, D), lambda i: (i, 0)),
            pl.BlockSpec((bn, 1), lambda i: (i, 0)),
            pl.BlockSpec((R, _DP, H), lambda i: (0, 0, 0)),
            pl.BlockSpec((D, H), lambda i: (0, 0)),
            pl.BlockSpec((1, H), lambda i: (0, 0)),
            pl.BlockSpec((H, D), lambda i: (0, 0)),
            pl.BlockSpec((1, D), lambda i: (0, 0)),
            pl.BlockSpec((D, D), lambda i: (0, 0)),
            pl.BlockSpec((1, D), lambda i: (0, 0)),
        ],
        out_specs=pl.BlockSpec((B, D), lambda i: (0, 0)),
        out_shape=jax.ShapeDtypeStruct((B, D), jnp.float32),
    )(acc3, x, batch_id.reshape(N, 1).astype(jnp.int32), W_rel_pad,
      W_root, b_conv.reshape(1, H), lin1_W, lin1_b.reshape(1, D),
      lin2_W, lin2_b.reshape(1, D))

    a_emb = jnp.take(ent_table, answers, axis=0)
    c_emb = jnp.take(ent_table, corrupted, axis=0)
    loss, golden, corrupt = pl.pallas_call(
        _score_kernel,
        out_shape=[jax.ShapeDtypeStruct((B, 1), jnp.float32)] * 3,
    )(q, a_emb, c_emb)
    return (loss.reshape(B), golden.reshape(B), corrupt.reshape(B))
